# Initial kernel scaffold; baseline (speedup 1.0000x reference)
import jax
import jax.numpy as jnp
from jax.experimental import pallas as pl


def kernel(x, edge_index, W1, b1, W2, b2, Wd1, bd1, Wd2, bd2):
    raise NotImplementedError("write your pallas kernel here")


# speedup vs baseline: 45.9387x; 45.9387x over previous
import functools

import jax
import jax.numpy as jnp
from jax import lax
from jax.experimental import pallas as pl
from jax.experimental.pallas import tpu as pltpu
from jax.experimental.pallas import tpu_sc as plsc

_N = 100000
_E = 1600000
_NW = 32
_SLC = 6272
_NPAD = 16 * _SLC
_RB = _NPAD // 128
_K = 8
_RPW = 392
_GROUPS = _RPW // _K
_EPAD = _NW * _RPW * 128
_ER = _EPAD // 128

_mesh = plsc.VectorSubcoreMesh(core_axis_name="c", subcore_axis_name="s")


@jax.jit
def _sc_deg(dst2d, zeros1):
  @functools.partial(
      pl.kernel,
      out_type=jax.ShapeDtypeStruct((2, _NPAD), jnp.float32),
      mesh=_mesh,
      scratch_types=[
          pltpu.VMEM((_K, 128), jnp.int32),
          pltpu.VMEM((128,), jnp.float32),
          pltpu.VMEM_SHARED((_NPAD,), jnp.float32),
      ],
  )
  def k(dst_hbm, z_hbm, out_hbm, idx_v, ones_v, acc_sh):
    c = lax.axis_index("c")
    s = lax.axis_index("s")
    wid = s * 2 + c
    for i in range(8):
      ones_v[pl.ds(i * 16, 16)] = jnp.ones((16,), jnp.float32)
    slc = pl.ds(s * _SLC, _SLC)
    pltpu.sync_copy(z_hbm.at[slc], acc_sh.at[slc])
    plsc.subcore_barrier()
    base = wid * _RPW

    @pl.loop(0, _GROUPS)
    def _(g):
      pltpu.sync_copy(dst_hbm.at[pl.ds(base + g * _K, _K)], idx_v)
      for j in range(_K):
        pltpu.sync_copy(ones_v, acc_sh.at[idx_v.at[j]], add=True)

    plsc.subcore_barrier()
    pltpu.sync_copy(acc_sh.at[slc], out_hbm.at[c].at[slc])

  return k(dst2d, zeros1)


@jax.jit
def _sc_edge1(src2d, dst2d, p1, zeros1):
  @functools.partial(
      pl.kernel,
      out_type=jax.ShapeDtypeStruct((2, _NPAD), jnp.float32),
      mesh=_mesh,
      scratch_types=[
          pltpu.VMEM((_K, 128), jnp.int32),
          pltpu.VMEM((_K, 128), jnp.int32),
          pltpu.VMEM((_K, 128), jnp.float32),
          pltpu.VMEM_SHARED((_NPAD,), jnp.float32),
          pltpu.VMEM_SHARED((_NPAD,), jnp.float32),
      ],
  )
  def k(src_hbm, dst_hbm, p1_hbm, z_hbm, out_hbm, idxs_v, idxd_v, vals_v,
        p1_sh, acc_sh):
    c = lax.axis_index("c")
    s = lax.axis_index("s")
    wid = s * 2 + c
    slc = pl.ds(s * _SLC, _SLC)
    pltpu.sync_copy(p1_hbm.at[slc], p1_sh.at[slc])
    pltpu.sync_copy(z_hbm.at[slc], acc_sh.at[slc])
    plsc.subcore_barrier()
    base = wid * _RPW

    @pl.loop(0, _GROUPS)
    def _(g):
      pltpu.sync_copy(src_hbm.at[pl.ds(base + g * _K, _K)], idxs_v)
      pltpu.sync_copy(dst_hbm.at[pl.ds(base + g * _K, _K)], idxd_v)
      for j in range(_K):
        pltpu.sync_copy(p1_sh.at[idxs_v.at[j]], vals_v.at[j])
      for j in range(_K):
        pltpu.sync_copy(vals_v.at[j], acc_sh.at[idxd_v.at[j]], add=True)

    plsc.subcore_barrier()
    pltpu.sync_copy(acc_sh.at[slc], out_hbm.at[c].at[slc])

  return k(src2d, dst2d, p1, zeros1)


@jax.jit
def _sc_edge4(src2d, dst2d, p2, zeros1):
  @functools.partial(
      pl.kernel,
      out_type=jax.ShapeDtypeStruct((2, 4, _NPAD), jnp.float32),
      mesh=_mesh,
      scratch_types=[
          pltpu.VMEM((_K, 128), jnp.int32),
          pltpu.VMEM((_K, 128), jnp.int32),
          pltpu.VMEM((_K, 128), jnp.float32),
          [pltpu.VMEM_SHARED((_NPAD,), jnp.float32) for _ in range(4)],
          [pltpu.VMEM_SHARED((_NPAD,), jnp.float32) for _ in range(4)],
      ],
  )
  def k(src_hbm, dst_hbm, p2_hbm, z_hbm, out_hbm, idxs_v, idxd_v, vals_v,
        p2_shs, acc_shs):
    c = lax.axis_index("c")
    s = lax.axis_index("s")
    wid = s * 2 + c
    slc = pl.ds(s * _SLC, _SLC)
    for f in range(4):
      pltpu.sync_copy(p2_hbm.at[f].at[slc], p2_shs[f].at[slc])
      pltpu.sync_copy(z_hbm.at[slc], acc_shs[f].at[slc])
    plsc.subcore_barrier()
    base = wid * _RPW

    for f in range(4):
      @pl.loop(0, _GROUPS)
      def _(g, f=f):
        pltpu.sync_copy(src_hbm.at[pl.ds(base + g * _K, _K)], idxs_v)
        pltpu.sync_copy(dst_hbm.at[pl.ds(base + g * _K, _K)], idxd_v)
        for j in range(_K):
          pltpu.sync_copy(p2_shs[f].at[idxs_v.at[j]], vals_v.at[j])
        for j in range(_K):
          pltpu.sync_copy(vals_v.at[j], acc_shs[f].at[idxd_v.at[j]], add=True)

    plsc.subcore_barrier()
    for f in range(4):
      pltpu.sync_copy(acc_shs[f].at[slc], out_hbm.at[c].at[f].at[slc])

  return k(src2d, dst2d, p2, zeros1)


def _tc_d1_body(deg_ref, x0_ref, dinv_ref, p1_ref):
  deg = deg_ref[0] + deg_ref[1] + 1.0
  dinv = lax.rsqrt(deg)
  dinv = dinv * (1.5 - 0.5 * deg * dinv * dinv)
  dinv_ref[...] = dinv
  p1_ref[...] = dinv * x0_ref[...]


@jax.jit
def _tc_d1(deg2, x0):
  return pl.pallas_call(
      _tc_d1_body,
      out_shape=(
          jax.ShapeDtypeStruct((_RB, 128), jnp.float32),
          jax.ShapeDtypeStruct((_RB, 128), jnp.float32),
      ),
  )(deg2, x0)


def _bf16r(v):
  return v.astype(jnp.bfloat16).astype(jnp.float32)


def _tc_d2_body(s1_ref, dinv_ref, x0_ref, w1_ref, b1_ref, w2_ref,
                hw_ref, p2_ref):
  dinv = dinv_ref[...]
  s1 = s1_ref[0] + s1_ref[1]
  a = dinv * (s1 + dinv * x0_ref[...])
  acc = [None] * 4
  for kk in range(32):
    t = _bf16r(jnp.maximum(a * w1_ref[0, kk] + b1_ref[0, kk], 0.0))
    for j in range(4):
      u = t * _bf16r(w2_ref[kk, j])
      acc[j] = u if acc[j] is None else acc[j] + u
  for j in range(4):
    hw_ref[j] = acc[j]
    p2_ref[j] = acc[j] * dinv


@jax.jit
def _tc_d2(s1_2, dinv, x0, w1, b1, w2):
  smem = pl.BlockSpec(memory_space=pltpu.SMEM)
  vmem = pl.BlockSpec(memory_space=pltpu.VMEM)
  return pl.pallas_call(
      _tc_d2_body,
      in_specs=[vmem, vmem, vmem, smem, smem, smem],
      out_shape=(
          jax.ShapeDtypeStruct((4, _RB, 128), jnp.float32),
          jax.ShapeDtypeStruct((4, _RB, 128), jnp.float32),
      ),
  )(s1_2, dinv, x0, w1, b1, w2)


def _tc_d3_body(s2_ref, dinv_ref, hw_ref, b2_ref, wd1_ref, bd1_ref,
                wd2_ref, bd2_ref, out_ref):
  dinv = dinv_ref[...]
  z = [None] * 4
  for j in range(4):
    zj = dinv * (s2_ref[0, j] + s2_ref[1, j] + dinv * hw_ref[j]) \
        + b2_ref[0, j]
    z[j] = _bf16r(zj)
  out = None
  for kk in range(32):
    t = z[0] * _bf16r(wd1_ref[0, kk])
    for j in range(1, 4):
      t = t + z[j] * _bf16r(wd1_ref[j, kk])
    t = _bf16r(jnp.maximum(t + bd1_ref[0, kk], 0.0))
    u = t * _bf16r(wd2_ref[kk, 0])
    out = u if out is None else out + u
  out_ref[...] = out + bd2_ref[0, 0]


@jax.jit
def _tc_d3(s2m, dinv, hw, b2, wd1, bd1, wd2, bd2):
  smem = pl.BlockSpec(memory_space=pltpu.SMEM)
  vmem = pl.BlockSpec(memory_space=pltpu.VMEM)
  return pl.pallas_call(
      _tc_d3_body,
      in_specs=[vmem, vmem, vmem, smem, smem, smem, smem, smem],
      out_shape=jax.ShapeDtypeStruct((_RB, 128), jnp.float32),
  )(s2m, dinv, hw, b2, wd1, bd1, wd2, bd2)


def kernel(x, edge_index, W1, b1, W2, b2, Wd1, bd1, Wd2, bd2):
  src = edge_index[0]
  dst = edge_index[1]
  npad_edges = _EPAD - _E
  padidx = _N + (jnp.arange(npad_edges, dtype=jnp.int32) % (_NPAD - _N))
  src2d = jnp.concatenate([src, padidx]).reshape(_ER, 128)
  dst2d = jnp.concatenate([dst, padidx]).reshape(_ER, 128)
  zeros1 = jnp.zeros((_NPAD,), jnp.float32)
  x0 = jnp.pad(x[:, 0], (0, _NPAD - _N)).reshape(_RB, 128)

  deg2 = _sc_deg(dst2d, zeros1)
  dinv, p1 = _tc_d1(deg2.reshape(2, _RB, 128), x0)
  s1_2 = _sc_edge1(src2d, dst2d, p1.reshape(_NPAD), zeros1)
  hw, p2m = _tc_d2(s1_2.reshape(2, _RB, 128), dinv, x0,
                   W1, b1.reshape(1, 32), W2)
  s2_2 = _sc_edge4(src2d, dst2d, p2m.reshape(4, _NPAD), zeros1)
  s2m = s2_2.reshape(2, 4, _RB, 128)
  xh = _tc_d3(s2m, dinv, hw, b2.reshape(1, 4),
              Wd1, bd1.reshape(1, 32), Wd2, bd2.reshape(1, 1))
  return xh.reshape(_NPAD)[:_N, None]


# speedup vs baseline: 51.4811x; 1.1206x over previous
import functools

import jax
import jax.numpy as jnp
from jax import lax
from jax.experimental import pallas as pl
from jax.experimental.pallas import tpu as pltpu
from jax.experimental.pallas import tpu_sc as plsc

_N = 100000
_E = 1600000
_NW = 32
_SLC = 6272
_NPAD = 16 * _SLC
_RB = _NPAD // 128
_K = 4
_RPW = 400
_PAIRS = _RPW // (2 * _K)
_EPAD = _NW * _RPW * 128
_ER = _EPAD // 128

_mesh = plsc.VectorSubcoreMesh(core_axis_name="c", subcore_axis_name="s")


@jax.jit
def _sc_deg(dst2d, zeros1):
  @functools.partial(
      pl.kernel,
      out_type=jax.ShapeDtypeStruct((2, _NPAD), jnp.float32),
      mesh=_mesh,
      scratch_types=[
          pltpu.VMEM((2, _K, 128), jnp.int32),
          pltpu.VMEM((128,), jnp.float32),
          pltpu.VMEM_SHARED((_NPAD,), jnp.float32),
          pltpu.SemaphoreType.DMA,
          pltpu.SemaphoreType.DMA,
      ],
  )
  def k(dst_hbm, z_hbm, out_hbm, idx_v, ones_v, acc_sh, sem0, sem1):
    sems = (sem0, sem1)
    c = lax.axis_index("c")
    s = lax.axis_index("s")
    wid = s * 2 + c
    for i in range(8):
      ones_v[pl.ds(i * 16, 16)] = jnp.ones((16,), jnp.float32)
    slc = pl.ds(s * _SLC, _SLC)
    pltpu.sync_copy(z_hbm.at[slc], acc_sh.at[slc])
    plsc.subcore_barrier()
    base = wid * _RPW

    @pl.loop(0, _PAIRS)
    def _(h):
      for p in range(2):
        @pl.when(h > 0)
        def _():
          for j in range(_K):
            pltpu.make_async_copy(
                ones_v, acc_sh.at[idx_v.at[p].at[j]], sems[p]).wait()
        g = 2 * h + p
        pltpu.sync_copy(dst_hbm.at[pl.ds(base + g * _K, _K)], idx_v.at[p])
        for j in range(_K):
          pltpu.async_copy(ones_v, acc_sh.at[idx_v.at[p].at[j]], sems[p],
                           add=True)

    for p in range(2):
      for j in range(_K):
        pltpu.make_async_copy(
            ones_v, acc_sh.at[idx_v.at[p].at[j]], sems[p]).wait()
    plsc.subcore_barrier()
    pltpu.sync_copy(acc_sh.at[slc], out_hbm.at[c].at[slc])

  return k(dst2d, zeros1)


def _edge_pipeline(src_hbm, dst_hbm, table_sh, acc_sh, idxs_v, idxd_v,
                   vals_v, semg, sems, base):
  @pl.loop(0, _PAIRS)
  def _(h):
    for p in range(2):
      @pl.when(h > 0)
      def _():
        for j in range(_K):
          pltpu.make_async_copy(
              vals_v.at[p].at[j], acc_sh.at[idxd_v.at[p].at[j]],
              sems[p]).wait()
      g = 2 * h + p
      pltpu.sync_copy(src_hbm.at[pl.ds(base + g * _K, _K)], idxs_v.at[p])
      pltpu.sync_copy(dst_hbm.at[pl.ds(base + g * _K, _K)], idxd_v.at[p])
      gds = [
          pltpu.async_copy(table_sh.at[idxs_v.at[p].at[j]],
                           vals_v.at[p].at[j], semg)
          for j in range(_K)
      ]
      for d in gds:
        d.wait()
      for j in range(_K):
        pltpu.async_copy(vals_v.at[p].at[j], acc_sh.at[idxd_v.at[p].at[j]],
                         sems[p], add=True)

  for p in range(2):
    for j in range(_K):
      pltpu.make_async_copy(
          vals_v.at[p].at[j], acc_sh.at[idxd_v.at[p].at[j]], sems[p]).wait()


@jax.jit
def _sc_edge1(src2d, dst2d, p1, zeros1):
  @functools.partial(
      pl.kernel,
      out_type=jax.ShapeDtypeStruct((2, _NPAD), jnp.float32),
      mesh=_mesh,
      scratch_types=[
          pltpu.VMEM((2, _K, 128), jnp.int32),
          pltpu.VMEM((2, _K, 128), jnp.int32),
          pltpu.VMEM((2, _K, 128), jnp.float32),
          pltpu.VMEM_SHARED((_NPAD,), jnp.float32),
          pltpu.VMEM_SHARED((_NPAD,), jnp.float32),
          pltpu.SemaphoreType.DMA,
          pltpu.SemaphoreType.DMA,
          pltpu.SemaphoreType.DMA,
      ],
  )
  def k(src_hbm, dst_hbm, p1_hbm, z_hbm, out_hbm, idxs_v, idxd_v, vals_v,
        p1_sh, acc_sh, semg, sem0, sem1):
    c = lax.axis_index("c")
    s = lax.axis_index("s")
    wid = s * 2 + c
    slc = pl.ds(s * _SLC, _SLC)
    pltpu.sync_copy(p1_hbm.at[slc], p1_sh.at[slc])
    pltpu.sync_copy(z_hbm.at[slc], acc_sh.at[slc])
    plsc.subcore_barrier()
    _edge_pipeline(src_hbm, dst_hbm, p1_sh, acc_sh, idxs_v, idxd_v, vals_v,
                   semg, (sem0, sem1), wid * _RPW)
    plsc.subcore_barrier()
    pltpu.sync_copy(acc_sh.at[slc], out_hbm.at[c].at[slc])

  return k(src2d, dst2d, p1, zeros1)


@jax.jit
def _sc_edge4(src2d, dst2d, p2, zeros1):
  @functools.partial(
      pl.kernel,
      out_type=jax.ShapeDtypeStruct((2, 4, _NPAD), jnp.float32),
      mesh=_mesh,
      scratch_types=[
          pltpu.VMEM((2, _K, 128), jnp.int32),
          pltpu.VMEM((2, _K, 128), jnp.int32),
          pltpu.VMEM((2, _K, 128), jnp.float32),
          [pltpu.VMEM_SHARED((_NPAD,), jnp.float32) for _ in range(4)],
          [pltpu.VMEM_SHARED((_NPAD,), jnp.float32) for _ in range(4)],
          pltpu.SemaphoreType.DMA,
          pltpu.SemaphoreType.DMA,
          pltpu.SemaphoreType.DMA,
      ],
  )
  def k(src_hbm, dst_hbm, p2_hbm, z_hbm, out_hbm, idxs_v, idxd_v, vals_v,
        p2_shs, acc_shs, semg, sem0, sem1):
    c = lax.axis_index("c")
    s = lax.axis_index("s")
    wid = s * 2 + c
    slc = pl.ds(s * _SLC, _SLC)
    for f in range(4):
      pltpu.sync_copy(p2_hbm.at[f].at[slc], p2_shs[f].at[slc])
      pltpu.sync_copy(z_hbm.at[slc], acc_shs[f].at[slc])
    plsc.subcore_barrier()
    for f in range(4):
      _edge_pipeline(src_hbm, dst_hbm, p2_shs[f], acc_shs[f], idxs_v,
                     idxd_v, vals_v, semg, (sem0, sem1), wid * _RPW)
    plsc.subcore_barrier()
    for f in range(4):
      pltpu.sync_copy(acc_shs[f].at[slc], out_hbm.at[c].at[f].at[slc])

  return k(src2d, dst2d, p2, zeros1)


def _tc_d1_body(deg_ref, x0_ref, dinv_ref, p1_ref):
  deg = deg_ref[0] + deg_ref[1] + 1.0
  dinv = lax.rsqrt(deg)
  dinv = dinv * (1.5 - 0.5 * deg * dinv * dinv)
  dinv_ref[...] = dinv
  p1_ref[...] = dinv * x0_ref[...]


@jax.jit
def _tc_d1(deg2, x0):
  return pl.pallas_call(
      _tc_d1_body,
      out_shape=(
          jax.ShapeDtypeStruct((_RB, 128), jnp.float32),
          jax.ShapeDtypeStruct((_RB, 128), jnp.float32),
      ),
  )(deg2, x0)


def _bf16r(v):
  return v.astype(jnp.bfloat16).astype(jnp.float32)


def _tc_d2_body(s1_ref, dinv_ref, x0_ref, w1_ref, b1_ref, w2_ref,
                hw_ref, p2_ref):
  dinv = dinv_ref[...]
  s1 = s1_ref[0] + s1_ref[1]
  a = dinv * (s1 + dinv * x0_ref[...])
  acc = [None] * 4
  for kk in range(32):
    t = _bf16r(jnp.maximum(a * w1_ref[0, kk] + b1_ref[0, kk], 0.0))
    for j in range(4):
      u = t * _bf16r(w2_ref[kk, j])
      acc[j] = u if acc[j] is None else acc[j] + u
  for j in range(4):
    hw_ref[j] = acc[j]
    p2_ref[j] = acc[j] * dinv


@jax.jit
def _tc_d2(s1_2, dinv, x0, w1, b1, w2):
  smem = pl.BlockSpec(memory_space=pltpu.SMEM)
  vmem = pl.BlockSpec(memory_space=pltpu.VMEM)
  return pl.pallas_call(
      _tc_d2_body,
      in_specs=[vmem, vmem, vmem, smem, smem, smem],
      out_shape=(
          jax.ShapeDtypeStruct((4, _RB, 128), jnp.float32),
          jax.ShapeDtypeStruct((4, _RB, 128), jnp.float32),
      ),
  )(s1_2, dinv, x0, w1, b1, w2)


def _tc_d3_body(s2_ref, dinv_ref, hw_ref, b2_ref, wd1_ref, bd1_ref,
                wd2_ref, bd2_ref, out_ref):
  dinv = dinv_ref[...]
  z = [None] * 4
  for j in range(4):
    zj = dinv * (s2_ref[0, j] + s2_ref[1, j] + dinv * hw_ref[j]) \
        + b2_ref[0, j]
    z[j] = _bf16r(zj)
  out = None
  for kk in range(32):
    t = z[0] * _bf16r(wd1_ref[0, kk])
    for j in range(1, 4):
      t = t + z[j] * _bf16r(wd1_ref[j, kk])
    t = _bf16r(jnp.maximum(t + bd1_ref[0, kk], 0.0))
    u = t * _bf16r(wd2_ref[kk, 0])
    out = u if out is None else out + u
  out_ref[...] = out + bd2_ref[0, 0]


@jax.jit
def _tc_d3(s2m, dinv, hw, b2, wd1, bd1, wd2, bd2):
  smem = pl.BlockSpec(memory_space=pltpu.SMEM)
  vmem = pl.BlockSpec(memory_space=pltpu.VMEM)
  return pl.pallas_call(
      _tc_d3_body,
      in_specs=[vmem, vmem, vmem, smem, smem, smem, smem, smem],
      out_shape=jax.ShapeDtypeStruct((_RB, 128), jnp.float32),
  )(s2m, dinv, hw, b2, wd1, bd1, wd2, bd2)


def kernel(x, edge_index, W1, b1, W2, b2, Wd1, bd1, Wd2, bd2):
  src = edge_index[0]
  dst = edge_index[1]
  npad_edges = _EPAD - _E
  padidx = _N + (jnp.arange(npad_edges, dtype=jnp.int32) % (_NPAD - _N))
  src2d = jnp.concatenate([src, padidx]).reshape(_ER, 128)
  dst2d = jnp.concatenate([dst, padidx]).reshape(_ER, 128)
  zeros1 = jnp.zeros((_NPAD,), jnp.float32)
  x0 = jnp.pad(x[:, 0], (0, _NPAD - _N)).reshape(_RB, 128)

  deg2 = _sc_deg(dst2d, zeros1)
  dinv, p1 = _tc_d1(deg2.reshape(2, _RB, 128), x0)
  s1_2 = _sc_edge1(src2d, dst2d, p1.reshape(_NPAD), zeros1)
  hw, p2m = _tc_d2(s1_2.reshape(2, _RB, 128), dinv, x0,
                   W1, b1.reshape(1, 32), W2)
  s2_2 = _sc_edge4(src2d, dst2d, p2m.reshape(4, _NPAD), zeros1)
  s2m = s2_2.reshape(2, 4, _RB, 128)
  xh = _tc_d3(s2m, dinv, hw, b2.reshape(1, 4),
              Wd1, bd1.reshape(1, 32), Wd2, bd2.reshape(1, 1))
  return xh.reshape(_NPAD)[:_N, None]


# speedup vs baseline: 53.1822x; 1.0330x over previous
import functools

import jax
import jax.numpy as jnp
from jax import lax
from jax.experimental import pallas as pl
from jax.experimental.pallas import tpu as pltpu
from jax.experimental.pallas import tpu_sc as plsc

_N = 100000
_E = 1600000
_NW = 32
_SLC = 6272
_NPAD = 16 * _SLC
_RB = _NPAD // 128
_K = 4
_RPW = 400
_PAIRS = _RPW // (2 * _K)
_EPAD = _NW * _RPW * 128
_ER = _EPAD // 128

_mesh = plsc.VectorSubcoreMesh(core_axis_name="c", subcore_axis_name="s")


@jax.jit
def _sc_deg(dst2d, zeros1):
  @functools.partial(
      pl.kernel,
      out_type=jax.ShapeDtypeStruct((2, _NPAD), jnp.float32),
      mesh=_mesh,
      scratch_types=[
          pltpu.VMEM((2, _K, 128), jnp.int32),
          pltpu.VMEM((128,), jnp.float32),
          pltpu.VMEM_SHARED((_NPAD,), jnp.float32),
          pltpu.SemaphoreType.DMA,
          pltpu.SemaphoreType.DMA,
      ],
  )
  def k(dst_hbm, z_hbm, out_hbm, idx_v, ones_v, acc_sh, sem0, sem1):
    sems = (sem0, sem1)
    c = lax.axis_index("c")
    s = lax.axis_index("s")
    wid = s * 2 + c
    for i in range(8):
      ones_v[pl.ds(i * 16, 16)] = jnp.ones((16,), jnp.float32)
    slc = pl.ds(s * _SLC, _SLC)
    pltpu.sync_copy(z_hbm.at[slc], acc_sh.at[slc])
    plsc.subcore_barrier()
    base = wid * _RPW

    @pl.loop(0, _PAIRS)
    def _(h):
      for p in range(2):
        @pl.when(h > 0)
        def _():
          for j in range(_K):
            pltpu.make_async_copy(
                ones_v, acc_sh.at[idx_v.at[p].at[j]], sems[p]).wait()
        g = 2 * h + p
        pltpu.sync_copy(dst_hbm.at[pl.ds(base + g * _K, _K)], idx_v.at[p])
        for j in range(_K):
          pltpu.async_copy(ones_v, acc_sh.at[idx_v.at[p].at[j]], sems[p],
                           add=True)

    for p in range(2):
      for j in range(_K):
        pltpu.make_async_copy(
            ones_v, acc_sh.at[idx_v.at[p].at[j]], sems[p]).wait()
    plsc.subcore_barrier()
    pltpu.sync_copy(acc_sh.at[slc], out_hbm.at[c].at[slc])

  return k(dst2d, zeros1)


def _edge_pipeline(src_hbm, dst_hbm, table_v, acc_sh, idxs_v, idxd_v,
                   vals_v, sems, base):
  @pl.loop(0, _PAIRS)
  def _(h):
    for p in range(2):
      @pl.when(h > 0)
      def _():
        for j in range(_K):
          pltpu.make_async_copy(
              vals_v.at[p].at[j], acc_sh.at[idxd_v.at[p].at[j]],
              sems[p]).wait()
      g = 2 * h + p
      pltpu.sync_copy(src_hbm.at[pl.ds(base + g * _K, _K)], idxs_v.at[p])
      pltpu.sync_copy(dst_hbm.at[pl.ds(base + g * _K, _K)], idxd_v.at[p])
      for j in range(_K):
        for t in range(8):
          i16 = idxs_v[p, j, pl.ds(t * 16, 16)]
          vals_v[p, j, pl.ds(t * 16, 16)] = plsc.load_gather(table_v, [i16])
      for j in range(_K):
        pltpu.async_copy(vals_v.at[p].at[j], acc_sh.at[idxd_v.at[p].at[j]],
                         sems[p], add=True)

  for p in range(2):
    for j in range(_K):
      pltpu.make_async_copy(
          vals_v.at[p].at[j], acc_sh.at[idxd_v.at[p].at[j]], sems[p]).wait()


@jax.jit
def _sc_edge1(src2d, dst2d, p1, zeros1):
  @functools.partial(
      pl.kernel,
      out_type=jax.ShapeDtypeStruct((2, _NPAD), jnp.float32),
      mesh=_mesh,
      compiler_params=pltpu.CompilerParams(needs_layout_passes=False),
      scratch_types=[
          pltpu.VMEM((2, _K, 128), jnp.int32),
          pltpu.VMEM((2, _K, 128), jnp.int32),
          pltpu.VMEM((2, _K, 128), jnp.float32),
          pltpu.VMEM((_NPAD,), jnp.float32),
          pltpu.VMEM_SHARED((_NPAD,), jnp.float32),
          pltpu.SemaphoreType.DMA,
          pltpu.SemaphoreType.DMA,
      ],
  )
  def k(src_hbm, dst_hbm, p1_hbm, z_hbm, out_hbm, idxs_v, idxd_v, vals_v,
        table_v, acc_sh, sem0, sem1):
    c = lax.axis_index("c")
    s = lax.axis_index("s")
    wid = s * 2 + c
    slc = pl.ds(s * _SLC, _SLC)
    pltpu.sync_copy(p1_hbm, table_v)
    pltpu.sync_copy(z_hbm.at[slc], acc_sh.at[slc])
    plsc.subcore_barrier()
    _edge_pipeline(src_hbm, dst_hbm, table_v, acc_sh, idxs_v, idxd_v, vals_v,
                   (sem0, sem1), wid * _RPW)
    plsc.subcore_barrier()
    pltpu.sync_copy(acc_sh.at[slc], out_hbm.at[c].at[slc])

  return k(src2d, dst2d, p1, zeros1)


@jax.jit
def _sc_edge4(src2d, dst2d, p2, zeros1):
  @functools.partial(
      pl.kernel,
      out_type=jax.ShapeDtypeStruct((2, 4, _NPAD), jnp.float32),
      mesh=_mesh,
      compiler_params=pltpu.CompilerParams(needs_layout_passes=False),
      scratch_types=[
          pltpu.VMEM((2, _K, 128), jnp.int32),
          pltpu.VMEM((2, _K, 128), jnp.int32),
          pltpu.VMEM((2, _K, 128), jnp.float32),
          pltpu.VMEM((_NPAD,), jnp.float32),
          [pltpu.VMEM_SHARED((_NPAD,), jnp.float32) for _ in range(4)],
          pltpu.SemaphoreType.DMA,
          pltpu.SemaphoreType.DMA,
      ],
  )
  def k(src_hbm, dst_hbm, p2_hbm, z_hbm, out_hbm, idxs_v, idxd_v, vals_v,
        table_v, acc_shs, sem0, sem1):
    c = lax.axis_index("c")
    s = lax.axis_index("s")
    wid = s * 2 + c
    slc = pl.ds(s * _SLC, _SLC)
    for f in range(4):
      pltpu.sync_copy(z_hbm.at[slc], acc_shs[f].at[slc])
    plsc.subcore_barrier()
    for f in range(4):
      pltpu.sync_copy(p2_hbm.at[f], table_v)
      _edge_pipeline(src_hbm, dst_hbm, table_v, acc_shs[f], idxs_v,
                     idxd_v, vals_v, (sem0, sem1), wid * _RPW)
    plsc.subcore_barrier()
    for f in range(4):
      pltpu.sync_copy(acc_shs[f].at[slc], out_hbm.at[c].at[f].at[slc])

  return k(src2d, dst2d, p2, zeros1)


def _tc_d1_body(deg_ref, x0_ref, dinv_ref, p1_ref):
  deg = deg_ref[0] + deg_ref[1] + 1.0
  dinv = lax.rsqrt(deg)
  dinv = dinv * (1.5 - 0.5 * deg * dinv * dinv)
  dinv_ref[...] = dinv
  p1_ref[...] = dinv * x0_ref[...]


@jax.jit
def _tc_d1(deg2, x0):
  return pl.pallas_call(
      _tc_d1_body,
      out_shape=(
          jax.ShapeDtypeStruct((_RB, 128), jnp.float32),
          jax.ShapeDtypeStruct((_RB, 128), jnp.float32),
      ),
  )(deg2, x0)


def _bf16r(v):
  return v.astype(jnp.bfloat16).astype(jnp.float32)


def _tc_d2_body(s1_ref, dinv_ref, x0_ref, w1_ref, b1_ref, w2_ref,
                hw_ref, p2_ref):
  dinv = dinv_ref[...]
  s1 = s1_ref[0] + s1_ref[1]
  a = dinv * (s1 + dinv * x0_ref[...])
  acc = [None] * 4
  for kk in range(32):
    t = _bf16r(jnp.maximum(a * w1_ref[0, kk] + b1_ref[0, kk], 0.0))
    for j in range(4):
      u = t * _bf16r(w2_ref[kk, j])
      acc[j] = u if acc[j] is None else acc[j] + u
  for j in range(4):
    hw_ref[j] = acc[j]
    p2_ref[j] = acc[j] * dinv


@jax.jit
def _tc_d2(s1_2, dinv, x0, w1, b1, w2):
  smem = pl.BlockSpec(memory_space=pltpu.SMEM)
  vmem = pl.BlockSpec(memory_space=pltpu.VMEM)
  return pl.pallas_call(
      _tc_d2_body,
      in_specs=[vmem, vmem, vmem, smem, smem, smem],
      out_shape=(
          jax.ShapeDtypeStruct((4, _RB, 128), jnp.float32),
          jax.ShapeDtypeStruct((4, _RB, 128), jnp.float32),
      ),
  )(s1_2, dinv, x0, w1, b1, w2)


def _tc_d3_body(s2_ref, dinv_ref, hw_ref, b2_ref, wd1_ref, bd1_ref,
                wd2_ref, bd2_ref, out_ref):
  dinv = dinv_ref[...]
  z = [None] * 4
  for j in range(4):
    zj = dinv * (s2_ref[0, j] + s2_ref[1, j] + dinv * hw_ref[j]) \
        + b2_ref[0, j]
    z[j] = _bf16r(zj)
  out = None
  for kk in range(32):
    t = z[0] * _bf16r(wd1_ref[0, kk])
    for j in range(1, 4):
      t = t + z[j] * _bf16r(wd1_ref[j, kk])
    t = _bf16r(jnp.maximum(t + bd1_ref[0, kk], 0.0))
    u = t * _bf16r(wd2_ref[kk, 0])
    out = u if out is None else out + u
  out_ref[...] = out + bd2_ref[0, 0]


@jax.jit
def _tc_d3(s2m, dinv, hw, b2, wd1, bd1, wd2, bd2):
  smem = pl.BlockSpec(memory_space=pltpu.SMEM)
  vmem = pl.BlockSpec(memory_space=pltpu.VMEM)
  return pl.pallas_call(
      _tc_d3_body,
      in_specs=[vmem, vmem, vmem, smem, smem, smem, smem, smem],
      out_shape=jax.ShapeDtypeStruct((_RB, 128), jnp.float32),
  )(s2m, dinv, hw, b2, wd1, bd1, wd2, bd2)


def kernel(x, edge_index, W1, b1, W2, b2, Wd1, bd1, Wd2, bd2):
  src = edge_index[0]
  dst = edge_index[1]
  npad_edges = _EPAD - _E
  padidx = _N + (jnp.arange(npad_edges, dtype=jnp.int32) % (_NPAD - _N))
  src2d = jnp.concatenate([src, padidx]).reshape(_ER, 128)
  dst2d = jnp.concatenate([dst, padidx]).reshape(_ER, 128)
  zeros1 = jnp.zeros((_NPAD,), jnp.float32)
  x0 = jnp.pad(x[:, 0], (0, _NPAD - _N)).reshape(_RB, 128)

  deg2 = _sc_deg(dst2d, zeros1)
  dinv, p1 = _tc_d1(deg2.reshape(2, _RB, 128), x0)
  s1_2 = _sc_edge1(src2d, dst2d, p1.reshape(_NPAD), zeros1)
  hw, p2m = _tc_d2(s1_2.reshape(2, _RB, 128), dinv, x0,
                   W1, b1.reshape(1, 32), W2)
  s2_2 = _sc_edge4(src2d, dst2d, p2m.reshape(4, _NPAD), zeros1)
  s2m = s2_2.reshape(2, 4, _RB, 128)
  xh = _tc_d3(s2m, dinv, hw, b2.reshape(1, 4),
              Wd1, bd1.reshape(1, 32), Wd2, bd2.reshape(1, 1))
  return xh.reshape(_NPAD)[:_N, None]


# speedup vs baseline: 86.2963x; 1.6227x over previous
import functools

import jax
import jax.numpy as jnp
from jax import lax
from jax.experimental import pallas as pl
from jax.experimental.pallas import tpu as pltpu
from jax.experimental.pallas import tpu_sc as plsc

_N = 100000
_E = 1600000
_NW = 32
_SLC = 6272
_NPAD = 16 * _SLC
_RB = _NPAD // 128
_K = 4
_RPW = 396
_STEPS = _RPW // _K
_TRIPLES = _STEPS // 3
_EPAD = _NW * _RPW * 128
_ER = _EPAD // 128

_mesh = plsc.VectorSubcoreMesh(core_axis_name="c", subcore_axis_name="s")


@jax.jit
def _sc_deg(dst2d, zeros1):
  @functools.partial(
      pl.kernel,
      out_type=jax.ShapeDtypeStruct((2, _NPAD), jnp.float32),
      mesh=_mesh,
      scratch_types=[
          pltpu.VMEM((3, _K, 128), jnp.int32),
          pltpu.VMEM((128,), jnp.float32),
          pltpu.VMEM_SHARED((_NPAD,), jnp.float32),
          pltpu.SemaphoreType.DMA,
          pltpu.SemaphoreType.DMA,
          pltpu.SemaphoreType.DMA,
          pltpu.SemaphoreType.DMA,
      ],
  )
  def k(dst_hbm, z_hbm, out_hbm, idx_v, ones_v, acc_sh, semi, sem0, sem1,
        sem2):
    sems = (sem0, sem1, sem2)
    c = lax.axis_index("c")
    s = lax.axis_index("s")
    wid = s * 2 + c
    for i in range(8):
      ones_v[pl.ds(i * 16, 16)] = jnp.ones((16,), jnp.float32)
    slc = pl.ds(s * _SLC, _SLC)
    pltpu.sync_copy(z_hbm.at[slc], acc_sh.at[slc])
    plsc.subcore_barrier()
    base = wid * _RPW
    pltpu.async_copy(dst_hbm.at[pl.ds(base, _K)], idx_v.at[0], semi)

    @pl.loop(0, _TRIPLES)
    def _(h):
      for i in range(3):
        q = i
        qn = (i + 1) % 3
        g = 3 * h + i

        def drain(qn=qn):
          for j in range(_K):
            pltpu.make_async_copy(
                ones_v, acc_sh.at[idx_v.at[qn].at[j]], sems[qn]).wait()
        if i < 2:
          pl.when(h > 0)(drain)
        else:
          drain()

        pltpu.make_async_copy(
            dst_hbm.at[pl.ds(base + g * _K, _K)], idx_v.at[q], semi).wait()

        def prefetch(g=g, qn=qn):
          pltpu.async_copy(
              dst_hbm.at[pl.ds(base + (g + 1) * _K, _K)], idx_v.at[qn],
              semi)
        if i == 2:
          pl.when(h < _TRIPLES - 1)(prefetch)
        else:
          prefetch()

        for j in range(_K):
          pltpu.async_copy(ones_v, acc_sh.at[idx_v.at[q].at[j]], sems[q],
                           add=True)

    for q in (1, 2):
      for j in range(_K):
        pltpu.make_async_copy(
            ones_v, acc_sh.at[idx_v.at[q].at[j]], sems[q]).wait()
    plsc.subcore_barrier()
    pltpu.sync_copy(acc_sh.at[slc], out_hbm.at[c].at[slc])

  return k(dst2d, zeros1)


def _edge_pipeline(src_hbm, dst_hbm, table_v, acc_sh, idxs_v, idxd_v,
                   vals_v, semi_s, semi_d, sems, base):
  pltpu.async_copy(src_hbm.at[pl.ds(base, _K)], idxs_v.at[0], semi_s)
  pltpu.async_copy(dst_hbm.at[pl.ds(base, _K)], idxd_v.at[0], semi_d)

  @pl.loop(0, _TRIPLES)
  def _(h):
    for i in range(3):
      q = i
      qn = (i + 1) % 3
      g = 3 * h + i

      def drain(qn=qn):
        for j in range(_K):
          pltpu.make_async_copy(
              vals_v.at[qn].at[j], acc_sh.at[idxd_v.at[qn].at[j]],
              sems[qn]).wait()
      if i < 2:
        pl.when(h > 0)(drain)
      else:
        drain()

      pltpu.make_async_copy(
          src_hbm.at[pl.ds(base + g * _K, _K)], idxs_v.at[q], semi_s).wait()
      pltpu.make_async_copy(
          dst_hbm.at[pl.ds(base + g * _K, _K)], idxd_v.at[q], semi_d).wait()

      def prefetch(g=g, qn=qn):
        pltpu.async_copy(
            src_hbm.at[pl.ds(base + (g + 1) * _K, _K)], idxs_v.at[qn],
            semi_s)
        pltpu.async_copy(
            dst_hbm.at[pl.ds(base + (g + 1) * _K, _K)], idxd_v.at[qn],
            semi_d)
      if i == 2:
        pl.when(h < _TRIPLES - 1)(prefetch)
      else:
        prefetch()

      for j in range(_K):
        for t in range(8):
          i16 = idxs_v[q, j, pl.ds(t * 16, 16)]
          vals_v[q, j, pl.ds(t * 16, 16)] = plsc.load_gather(table_v, [i16])
      for j in range(_K):
        pltpu.async_copy(vals_v.at[q].at[j], acc_sh.at[idxd_v.at[q].at[j]],
                         sems[q], add=True)

  for q in (1, 2):
    for j in range(_K):
      pltpu.make_async_copy(
          vals_v.at[q].at[j], acc_sh.at[idxd_v.at[q].at[j]], sems[q]).wait()


@jax.jit
def _sc_edge1(src2d, dst2d, p1, zeros1):
  @functools.partial(
      pl.kernel,
      out_type=jax.ShapeDtypeStruct((2, _NPAD), jnp.float32),
      mesh=_mesh,
      compiler_params=pltpu.CompilerParams(needs_layout_passes=False),
      scratch_types=[
          pltpu.VMEM((3, _K, 128), jnp.int32),
          pltpu.VMEM((3, _K, 128), jnp.int32),
          pltpu.VMEM((3, _K, 128), jnp.float32),
          pltpu.VMEM((_NPAD,), jnp.float32),
          pltpu.VMEM_SHARED((_NPAD,), jnp.float32),
          pltpu.SemaphoreType.DMA,
          pltpu.SemaphoreType.DMA,
          pltpu.SemaphoreType.DMA,
          pltpu.SemaphoreType.DMA,
          pltpu.SemaphoreType.DMA,
      ],
  )
  def k(src_hbm, dst_hbm, p1_hbm, z_hbm, out_hbm, idxs_v, idxd_v, vals_v,
        table_v, acc_sh, semi_s, semi_d, sem0, sem1, sem2):
    c = lax.axis_index("c")
    s = lax.axis_index("s")
    wid = s * 2 + c
    slc = pl.ds(s * _SLC, _SLC)
    pltpu.sync_copy(p1_hbm, table_v)
    pltpu.sync_copy(z_hbm.at[slc], acc_sh.at[slc])
    plsc.subcore_barrier()
    _edge_pipeline(src_hbm, dst_hbm, table_v, acc_sh, idxs_v, idxd_v, vals_v,
                   semi_s, semi_d, (sem0, sem1, sem2), wid * _RPW)
    plsc.subcore_barrier()
    pltpu.sync_copy(acc_sh.at[slc], out_hbm.at[c].at[slc])

  return k(src2d, dst2d, p1, zeros1)


@jax.jit
def _sc_edge4(src2d, dst2d, p2, zeros1):
  @functools.partial(
      pl.kernel,
      out_type=jax.ShapeDtypeStruct((2, 4, _NPAD), jnp.float32),
      mesh=_mesh,
      compiler_params=pltpu.CompilerParams(needs_layout_passes=False),
      scratch_types=[
          pltpu.VMEM((3, _K, 128), jnp.int32),
          pltpu.VMEM((3, _K, 128), jnp.int32),
          pltpu.VMEM((3, _K, 128), jnp.float32),
          pltpu.VMEM((_NPAD,), jnp.float32),
          [pltpu.VMEM_SHARED((_NPAD,), jnp.float32) for _ in range(4)],
          pltpu.SemaphoreType.DMA,
          pltpu.SemaphoreType.DMA,
          pltpu.SemaphoreType.DMA,
          pltpu.SemaphoreType.DMA,
          pltpu.SemaphoreType.DMA,
      ],
  )
  def k(src_hbm, dst_hbm, p2_hbm, z_hbm, out_hbm, idxs_v, idxd_v, vals_v,
        table_v, acc_shs, semi_s, semi_d, sem0, sem1, sem2):
    c = lax.axis_index("c")
    s = lax.axis_index("s")
    wid = s * 2 + c
    slc = pl.ds(s * _SLC, _SLC)
    for f in range(4):
      pltpu.sync_copy(z_hbm.at[slc], acc_shs[f].at[slc])
    plsc.subcore_barrier()
    for f in range(4):
      pltpu.sync_copy(p2_hbm.at[f], table_v)
      _edge_pipeline(src_hbm, dst_hbm, table_v, acc_shs[f], idxs_v,
                     idxd_v, vals_v, semi_s, semi_d, (sem0, sem1, sem2),
                     wid * _RPW)
    plsc.subcore_barrier()
    for f in range(4):
      pltpu.sync_copy(acc_shs[f].at[slc], out_hbm.at[c].at[f].at[slc])

  return k(src2d, dst2d, p2, zeros1)


def _tc_d1_body(deg_ref, x0_ref, dinv_ref, p1_ref):
  deg = deg_ref[0] + deg_ref[1] + 1.0
  dinv = lax.rsqrt(deg)
  dinv = dinv * (1.5 - 0.5 * deg * dinv * dinv)
  dinv_ref[...] = dinv
  p1_ref[...] = dinv * x0_ref[...]


@jax.jit
def _tc_d1(deg2, x0):
  return pl.pallas_call(
      _tc_d1_body,
      out_shape=(
          jax.ShapeDtypeStruct((_RB, 128), jnp.float32),
          jax.ShapeDtypeStruct((_RB, 128), jnp.float32),
      ),
  )(deg2, x0)


def _bf16r(v):
  return v.astype(jnp.bfloat16).astype(jnp.float32)


def _tc_d2_body(s1_ref, dinv_ref, x0_ref, w1_ref, b1_ref, w2_ref,
                hw_ref, p2_ref):
  dinv = dinv_ref[...]
  s1 = s1_ref[0] + s1_ref[1]
  a = dinv * (s1 + dinv * x0_ref[...])
  acc = [None] * 4
  for kk in range(32):
    t = _bf16r(jnp.maximum(a * w1_ref[0, kk] + b1_ref[0, kk], 0.0))
    for j in range(4):
      u = t * _bf16r(w2_ref[kk, j])
      acc[j] = u if acc[j] is None else acc[j] + u
  for j in range(4):
    hw_ref[j] = acc[j]
    p2_ref[j] = acc[j] * dinv


@jax.jit
def _tc_d2(s1_2, dinv, x0, w1, b1, w2):
  smem = pl.BlockSpec(memory_space=pltpu.SMEM)
  vmem = pl.BlockSpec(memory_space=pltpu.VMEM)
  return pl.pallas_call(
      _tc_d2_body,
      in_specs=[vmem, vmem, vmem, smem, smem, smem],
      out_shape=(
          jax.ShapeDtypeStruct((4, _RB, 128), jnp.float32),
          jax.ShapeDtypeStruct((4, _RB, 128), jnp.float32),
      ),
  )(s1_2, dinv, x0, w1, b1, w2)


def _tc_d3_body(s2_ref, dinv_ref, hw_ref, b2_ref, wd1_ref, bd1_ref,
                wd2_ref, bd2_ref, out_ref):
  dinv = dinv_ref[...]
  z = [None] * 4
  for j in range(4):
    zj = dinv * (s2_ref[0, j] + s2_ref[1, j] + dinv * hw_ref[j]) \
        + b2_ref[0, j]
    z[j] = _bf16r(zj)
  out = None
  for kk in range(32):
    t = z[0] * _bf16r(wd1_ref[0, kk])
    for j in range(1, 4):
      t = t + z[j] * _bf16r(wd1_ref[j, kk])
    t = _bf16r(jnp.maximum(t + bd1_ref[0, kk], 0.0))
    u = t * _bf16r(wd2_ref[kk, 0])
    out = u if out is None else out + u
  out_ref[...] = out + bd2_ref[0, 0]


@jax.jit
def _tc_d3(s2m, dinv, hw, b2, wd1, bd1, wd2, bd2):
  smem = pl.BlockSpec(memory_space=pltpu.SMEM)
  vmem = pl.BlockSpec(memory_space=pltpu.VMEM)
  return pl.pallas_call(
      _tc_d3_body,
      in_specs=[vmem, vmem, vmem, smem, smem, smem, smem, smem],
      out_shape=jax.ShapeDtypeStruct((_RB, 128), jnp.float32),
  )(s2m, dinv, hw, b2, wd1, bd1, wd2, bd2)


def kernel(x, edge_index, W1, b1, W2, b2, Wd1, bd1, Wd2, bd2):
  src = edge_index[0]
  dst = edge_index[1]
  npad_edges = _EPAD - _E
  padidx = _N + (jnp.arange(npad_edges, dtype=jnp.int32) % (_NPAD - _N))
  src2d = jnp.concatenate([src, padidx]).reshape(_ER, 128)
  dst2d = jnp.concatenate([dst, padidx]).reshape(_ER, 128)
  zeros1 = jnp.zeros((_NPAD,), jnp.float32)
  x0 = jnp.pad(x[:, 0], (0, _NPAD - _N)).reshape(_RB, 128)

  deg2 = _sc_deg(dst2d, zeros1)
  dinv, p1 = _tc_d1(deg2.reshape(2, _RB, 128), x0)
  s1_2 = _sc_edge1(src2d, dst2d, p1.reshape(_NPAD), zeros1)
  hw, p2m = _tc_d2(s1_2.reshape(2, _RB, 128), dinv, x0,
                   W1, b1.reshape(1, 32), W2)
  s2_2 = _sc_edge4(src2d, dst2d, p2m.reshape(4, _NPAD), zeros1)
  s2m = s2_2.reshape(2, 4, _RB, 128)
  xh = _tc_d3(s2m, dinv, hw, b2.reshape(1, 4),
              Wd1, bd1.reshape(1, 32), Wd2, bd2.reshape(1, 1))
  return xh.reshape(_NPAD)[:_N, None]
